# baseline (speedup 1.0000x reference)
import functools

import jax
import jax.numpy as jnp
from jax import lax
from jax.experimental import pallas as pl
from jax.experimental.pallas import tpu as pltpu
from jax.experimental.pallas import tpu_sc as plsc

N = 10000
E = 320000
H = 128
G = 64

NC = 2
NS = 16
NW = NC * NS
EW = E // NW
CH = 80
NCH = EW // CH
GR = 5
NG = NCH // GR
NP = 10240
RT = NP // NS
ZR = 32


def _edge_body(h_hbm, src_hbm, dst_hbm, zeros_hbm, out_hbm, srcb, dstb,
               bank0, bank1, agg_sh, gsem0, gsem1, ssem0, ssem1):
    c = lax.axis_index("c")
    s = lax.axis_index("s")
    wid = s * NC + c

    base = pl.multiple_of(wid * EW, 8)
    pltpu.sync_copy(src_hbm.at[pl.ds(base, EW)], srcb)
    pltpu.sync_copy(dst_hbm.at[wid], dstb)

    row0 = pl.multiple_of(s * RT, 8)
    pltpu.sync_copy(zeros_hbm.at[pl.ds(row0, RT)], agg_sh.at[pl.ds(row0, RT)])
    plsc.subcore_barrier()

    def gather(j, bank, gsem):
        off = pl.multiple_of(j * CH, 8)
        pltpu.async_copy(h_hbm.at[srcb.at[pl.ds(off, CH)]], bank, gsem)

    def gwait(bank, gsem):
        pltpu.make_async_copy(h_hbm.at[srcb.at[pl.ds(0, CH)]], bank,
                              gsem).wait()

    def scatter(j, bank, ssem):
        pltpu.async_copy(bank, agg_sh.at[dstb.at[j]], ssem, add=True)

    def swait(bank, ssem):
        pltpu.make_async_copy(bank, agg_sh.at[dstb.at[0]], ssem).wait()

    gather(0, bank0, gsem0)
    gather(1, bank1, gsem1)

    def pipe_step(j2, _):
        ja = j2 * 2
        gwait(bank0, gsem0)
        scatter(ja, bank0, ssem0)
        gwait(bank1, gsem1)
        scatter(ja + 1, bank1, ssem1)
        swait(bank0, ssem0)
        gather(ja + 2, bank0, gsem0)
        swait(bank1, ssem1)

        @pl.when(j2 < (NCH - 3) // 2)
        def _():
            gather(ja + 3, bank1, gsem1)
        return 0

    lax.fori_loop(0, (NCH - 1) // 2, pipe_step, 0)
    gwait(bank0, gsem0)
    pltpu.sync_copy(bank0, agg_sh.at[dstb.at[NCH - 1]], add=True)
    plsc.subcore_barrier()

    row0 = pl.multiple_of(s * RT, 8)
    pltpu.sync_copy(agg_sh.at[pl.ds(row0, RT)],
                    out_hbm.at[c].at[pl.ds(row0, RT)])


_edge_agg = pl.kernel(
    _edge_body,
    out_type=jax.ShapeDtypeStruct((NC, NP, H), jnp.float32),
    mesh=plsc.VectorSubcoreMesh(core_axis_name="c", subcore_axis_name="s"),
    scratch_types=[
        pltpu.VMEM((EW,), jnp.int32),
        pltpu.VMEM((NCH, CH), jnp.int32),
        pltpu.VMEM((CH, H), jnp.float32),
        pltpu.VMEM((CH, H), jnp.float32),
        pltpu.VMEM_SHARED((NP, H), jnp.float32),
        pltpu.SemaphoreType.DMA,
        pltpu.SemaphoreType.DMA,
        pltpu.SemaphoreType.DMA,
        pltpu.SemaphoreType.DMA,
    ],
)


def _embed_body(x_ref, w_ref, b_ref, h_ref, r_ref):
    h = jnp.dot(x_ref[...], w_ref[...], preferred_element_type=jnp.float32,
                precision=lax.Precision.HIGHEST)
    h = h + b_ref[...]
    h_ref[...] = h
    r_ref[...] = jnp.maximum(h, 0.0)


def _embed(x, W_ln, b_ln):
    return pl.pallas_call(
        _embed_body,
        out_shape=[
            jax.ShapeDtypeStruct((N, H), jnp.float32),
            jax.ShapeDtypeStruct((N, H), jnp.float32),
        ],
    )(x, W_ln, b_ln.reshape(1, H))


def _bn_cols(u, g, b):
    m = jnp.mean(u, axis=0, keepdims=True)
    v = jnp.mean((u - m) ** 2, axis=0, keepdims=True)
    return (u - m) / jnp.sqrt(v + 1e-5) * g + b


def _mlp_body(last, h_ref, agg_ref, w1_ref, b1_ref, g1_ref, be1_ref,
              w2_ref, b2_ref, eps_ref, bng_ref, bnb_ref, out_ref):
    agg = agg_ref[0, :N] + agg_ref[1, :N]
    t = h_ref[...] * (1.0 + eps_ref[0, 0]) + agg
    u = jnp.dot(t, w1_ref[...], preferred_element_type=jnp.float32,
                precision=lax.Precision.HIGHEST)
    u = u + b1_ref[...]
    u = _bn_cols(u, g1_ref[...], be1_ref[...])
    u = jnp.maximum(u, 0.0)
    o = jnp.dot(u, w2_ref[...], preferred_element_type=jnp.float32,
                precision=lax.Precision.HIGHEST)
    o = o + b2_ref[...]
    if not last:
        o = _bn_cols(o, bng_ref[...], bnb_ref[...])
        o = jnp.maximum(o, 0.0)
    out_ref[...] = o


def _mlp(h, agg, w1, b1, g1, be1, w2, b2, eps_i, bng_i, bnb_i, last):
    return pl.pallas_call(
        functools.partial(_mlp_body, last),
        out_shape=jax.ShapeDtypeStruct((N, H), jnp.float32),
    )(h, agg, w1, b1.reshape(1, -1), g1.reshape(1, -1), be1.reshape(1, -1),
      w2, b2.reshape(1, -1), eps_i.reshape(1, 1), bng_i.reshape(1, -1),
      bnb_i.reshape(1, -1))


def _pool_body(h_ref, batch_ref, wo_ref, bo_ref, out_ref):
    gids = lax.broadcasted_iota(jnp.int32, (G, N), 0)
    onehot = (batch_ref[...] == gids).astype(jnp.float32)
    sums = jnp.dot(onehot, h_ref[...], preferred_element_type=jnp.float32,
                precision=lax.Precision.HIGHEST)
    counts = jnp.maximum(jnp.sum(onehot, axis=1, keepdims=True), 1.0)
    pooled = sums / counts
    out_ref[...] = (
        jnp.dot(pooled, wo_ref[...], preferred_element_type=jnp.float32,
                precision=lax.Precision.HIGHEST)
        + bo_ref[...])


def _pool(h, batch, W_out, b_out):
    return pl.pallas_call(
        _pool_body,
        out_shape=jax.ShapeDtypeStruct((G, W_out.shape[1]), jnp.float32),
    )(h, batch.reshape(1, N), W_out, b_out.reshape(1, -1))


def kernel(x, edge_attr, W_ln, b_ln, W1, b1, g1, be1, W2, b2, eps, bng, bnb,
           W_out, b_out, edge_index, batch):
    src = edge_index[0]
    dst = edge_index[1].reshape(NW, NCH, CH)
    zeros = jnp.zeros((NP, H), jnp.float32)
    h, r = _embed(x, W_ln, b_ln)
    L = W1.shape[0]
    for i in range(L):
        agg = _edge_agg(r if i == 0 else h, src, dst, zeros)
        j = min(i, L - 2)
        h = _mlp(h, agg, W1[i], b1[i], g1[i], be1[i], W2[i], b2[i],
                 eps[i], bng[j], bnb[j], last=(i == L - 1))
    return _pool(h, batch, W_out, b_out)


# speedup vs baseline: 1.1930x; 1.1930x over previous
import functools

import jax
import jax.numpy as jnp
from jax import lax
from jax.experimental import pallas as pl
from jax.experimental.pallas import tpu as pltpu
from jax.experimental.pallas import tpu_sc as plsc

N = 10000
E = 320000
H = 128
G = 64

NC = 2
NS = 16
NW = NC * NS
EW = E // NW
CH = 80
NCH = EW // CH
GR = 5
NG = NCH // GR
NP = 10240
RT = NP // NS
ZR = 32


def _edge_body(h_hbm, src_hbm, dst_hbm, zeros_hbm, out_hbm, srcb, dstb,
               bank0, bank1, agg_sh, gsem0, gsem1):
    c = lax.axis_index("c")
    s = lax.axis_index("s")
    wid = s * NC + c

    base = pl.multiple_of(wid * EW, 8)
    pltpu.sync_copy(src_hbm.at[pl.ds(base, EW)], srcb)
    pltpu.sync_copy(dst_hbm.at[wid], dstb)

    row0 = pl.multiple_of(s * RT, 8)
    pltpu.sync_copy(zeros_hbm.at[pl.ds(row0, RT)], agg_sh.at[pl.ds(row0, RT)])
    plsc.subcore_barrier()

    def gather(j, bank, gsem):
        off = pl.multiple_of(j * CH, 8)
        pltpu.async_copy(h_hbm.at[srcb.at[pl.ds(off, CH)]], bank, gsem)

    def gwait(bank, gsem):
        pltpu.make_async_copy(h_hbm.at[srcb.at[pl.ds(0, CH)]], bank,
                              gsem).wait()

    def scatter(j, bank):
        pltpu.sync_copy(bank, agg_sh.at[dstb.at[j]], add=True)

    gather(0, bank0, gsem0)
    gather(1, bank1, gsem1)

    def pipe_step(j2, _):
        ja = j2 * 2
        gwait(bank0, gsem0)
        scatter(ja, bank0)
        gather(ja + 2, bank0, gsem0)
        gwait(bank1, gsem1)
        scatter(ja + 1, bank1)

        @pl.when(j2 < (NCH - 3) // 2)
        def _():
            gather(ja + 3, bank1, gsem1)
        return 0

    lax.fori_loop(0, (NCH - 1) // 2, pipe_step, 0)
    gwait(bank0, gsem0)
    scatter(NCH - 1, bank0)
    plsc.subcore_barrier()

    row0 = pl.multiple_of(s * RT, 8)
    pltpu.sync_copy(agg_sh.at[pl.ds(row0, RT)],
                    out_hbm.at[c].at[pl.ds(row0, RT)])


_edge_agg = pl.kernel(
    _edge_body,
    out_type=jax.ShapeDtypeStruct((NC, NP, H), jnp.float32),
    mesh=plsc.VectorSubcoreMesh(core_axis_name="c", subcore_axis_name="s"),
    scratch_types=[
        pltpu.VMEM((EW,), jnp.int32),
        pltpu.VMEM((NCH, CH), jnp.int32),
        pltpu.VMEM((CH, H), jnp.float32),
        pltpu.VMEM((CH, H), jnp.float32),
        pltpu.VMEM_SHARED((NP, H), jnp.float32),
        pltpu.SemaphoreType.DMA,
        pltpu.SemaphoreType.DMA,
    ],
)


def _embed_body(x_ref, w_ref, b_ref, h_ref, r_ref):
    h = jnp.dot(x_ref[...], w_ref[...], preferred_element_type=jnp.float32,
                precision=lax.Precision.HIGHEST)
    h = h + b_ref[...]
    h_ref[...] = h
    r_ref[...] = jnp.maximum(h, 0.0)


def _embed(x, W_ln, b_ln):
    return pl.pallas_call(
        _embed_body,
        out_shape=[
            jax.ShapeDtypeStruct((N, H), jnp.float32),
            jax.ShapeDtypeStruct((N, H), jnp.float32),
        ],
    )(x, W_ln, b_ln.reshape(1, H))


def _bn_cols(u, g, b):
    m = jnp.mean(u, axis=0, keepdims=True)
    v = jnp.mean((u - m) ** 2, axis=0, keepdims=True)
    return (u - m) / jnp.sqrt(v + 1e-5) * g + b


def _mlp_body(last, h_ref, agg_ref, w1_ref, b1_ref, g1_ref, be1_ref,
              w2_ref, b2_ref, eps_ref, bng_ref, bnb_ref, out_ref):
    agg = agg_ref[0, :N] + agg_ref[1, :N]
    t = h_ref[...] * (1.0 + eps_ref[0, 0]) + agg
    u = jnp.dot(t, w1_ref[...], preferred_element_type=jnp.float32,
                precision=lax.Precision.HIGHEST)
    u = u + b1_ref[...]
    u = _bn_cols(u, g1_ref[...], be1_ref[...])
    u = jnp.maximum(u, 0.0)
    o = jnp.dot(u, w2_ref[...], preferred_element_type=jnp.float32,
                precision=lax.Precision.HIGHEST)
    o = o + b2_ref[...]
    if not last:
        o = _bn_cols(o, bng_ref[...], bnb_ref[...])
        o = jnp.maximum(o, 0.0)
    out_ref[...] = o


def _mlp(h, agg, w1, b1, g1, be1, w2, b2, eps_i, bng_i, bnb_i, last):
    return pl.pallas_call(
        functools.partial(_mlp_body, last),
        out_shape=jax.ShapeDtypeStruct((N, H), jnp.float32),
    )(h, agg, w1, b1.reshape(1, -1), g1.reshape(1, -1), be1.reshape(1, -1),
      w2, b2.reshape(1, -1), eps_i.reshape(1, 1), bng_i.reshape(1, -1),
      bnb_i.reshape(1, -1))


def _pool_body(h_ref, batch_ref, wo_ref, bo_ref, out_ref):
    gids = lax.broadcasted_iota(jnp.int32, (G, N), 0)
    onehot = (batch_ref[...] == gids).astype(jnp.float32)
    sums = jnp.dot(onehot, h_ref[...], preferred_element_type=jnp.float32,
                precision=lax.Precision.HIGHEST)
    counts = jnp.maximum(jnp.sum(onehot, axis=1, keepdims=True), 1.0)
    pooled = sums / counts
    out_ref[...] = (
        jnp.dot(pooled, wo_ref[...], preferred_element_type=jnp.float32,
                precision=lax.Precision.HIGHEST)
        + bo_ref[...])


def _pool(h, batch, W_out, b_out):
    return pl.pallas_call(
        _pool_body,
        out_shape=jax.ShapeDtypeStruct((G, W_out.shape[1]), jnp.float32),
    )(h, batch.reshape(1, N), W_out, b_out.reshape(1, -1))


def kernel(x, edge_attr, W_ln, b_ln, W1, b1, g1, be1, W2, b2, eps, bng, bnb,
           W_out, b_out, edge_index, batch):
    src = edge_index[0]
    dst = edge_index[1].reshape(NW, NCH, CH)
    zeros = jnp.zeros((NP, H), jnp.float32)
    h, r = _embed(x, W_ln, b_ln)
    L = W1.shape[0]
    for i in range(L):
        agg = _edge_agg(r if i == 0 else h, src, dst, zeros)
        j = min(i, L - 2)
        h = _mlp(h, agg, W1[i], b1[i], g1[i], be1[i], W2[i], b2[i],
                 eps[i], bng[j], bnb[j], last=(i == L - 1))
    return _pool(h, batch, W_out, b_out)


# speedup vs baseline: 1.3386x; 1.1220x over previous
import functools

import jax
import jax.numpy as jnp
from jax import lax
from jax.experimental import pallas as pl
from jax.experimental.pallas import tpu as pltpu
from jax.experimental.pallas import tpu_sc as plsc

N = 10000
E = 320000
H = 128
G = 64

NC = 2
NS = 16
NW = NC * NS
EW = E // NW
CH = 80
NCH = EW // CH
GR = 5
NG = NCH // GR
NP = 10240
RT = NP // NS
ZR = 32


def _edge_body(h_hbm, src_hbm, dst_hbm, zeros_hbm, out_hbm, srcb, dstb,
               bank0, bank1, agg_sh, gsem0, gsem1):
    c = lax.axis_index("c")
    s = lax.axis_index("s")
    wid = s * NC + c

    base = pl.multiple_of(wid * EW, 8)
    pltpu.sync_copy(src_hbm.at[pl.ds(base, EW)], srcb)
    pltpu.sync_copy(dst_hbm.at[wid], dstb)

    def gather(j, bank, gsem):
        off = pl.multiple_of(j * CH, 8)
        pltpu.async_copy(h_hbm.at[srcb.at[pl.ds(off, CH)]], bank, gsem)

    def gwait(bank, gsem):
        pltpu.make_async_copy(h_hbm.at[srcb.at[pl.ds(0, CH)]], bank,
                              gsem).wait()

    def scatter(j, bank):
        pltpu.sync_copy(bank, agg_sh.at[dstb.at[j]], add=True)

    gather(0, bank0, gsem0)
    gather(1, bank1, gsem1)
    row0 = pl.multiple_of(s * RT, 8)
    pltpu.sync_copy(zeros_hbm.at[pl.ds(row0, RT)], agg_sh.at[pl.ds(row0, RT)])
    plsc.subcore_barrier()

    def pipe_step(j2, _):
        ja = j2 * 2
        gwait(bank0, gsem0)
        scatter(ja, bank0)
        gather(ja + 2, bank0, gsem0)
        gwait(bank1, gsem1)
        scatter(ja + 1, bank1)

        @pl.when(j2 < (NCH - 3) // 2)
        def _():
            gather(ja + 3, bank1, gsem1)
        return 0

    lax.fori_loop(0, (NCH - 1) // 2, pipe_step, 0)
    gwait(bank0, gsem0)
    scatter(NCH - 1, bank0)
    plsc.subcore_barrier()

    row0 = pl.multiple_of(s * RT, 8)
    pltpu.sync_copy(agg_sh.at[pl.ds(row0, RT)],
                    out_hbm.at[c].at[pl.ds(row0, RT)])


_edge_agg = pl.kernel(
    _edge_body,
    out_type=jax.ShapeDtypeStruct((NC, NP, H), jnp.float32),
    mesh=plsc.VectorSubcoreMesh(core_axis_name="c", subcore_axis_name="s"),
    scratch_types=[
        pltpu.VMEM((EW,), jnp.int32),
        pltpu.VMEM((NCH, CH), jnp.int32),
        pltpu.VMEM((CH, H), jnp.float32),
        pltpu.VMEM((CH, H), jnp.float32),
        pltpu.VMEM_SHARED((NP, H), jnp.float32),
        pltpu.SemaphoreType.DMA,
        pltpu.SemaphoreType.DMA,
    ],
)


def _embed_body(x_ref, w_ref, b_ref, h_ref, r_ref):
    h = _dot3(x_ref[...], w_ref[...])
    h = h + b_ref[...]
    h_ref[...] = h
    r_ref[...] = jnp.maximum(h, 0.0)


def _embed(x, W_ln, b_ln):
    return pl.pallas_call(
        _embed_body,
        out_shape=[
            jax.ShapeDtypeStruct((N, H), jnp.float32),
            jax.ShapeDtypeStruct((N, H), jnp.float32),
        ],
    )(x, W_ln, b_ln.reshape(1, H))


def _dot3(a, b):
    a_hi = a.astype(jnp.bfloat16)
    a_lo = (a - a_hi.astype(jnp.float32)).astype(jnp.bfloat16)
    b_hi = b.astype(jnp.bfloat16)
    b_lo = (b - b_hi.astype(jnp.float32)).astype(jnp.bfloat16)
    f = jnp.float32
    return (jnp.dot(a_hi, b_hi, preferred_element_type=f)
            + jnp.dot(a_hi, b_lo, preferred_element_type=f)
            + jnp.dot(a_lo, b_hi, preferred_element_type=f))


def _bn_cols(u, g, b):
    m = jnp.mean(u, axis=0, keepdims=True)
    v = jnp.mean((u - m) ** 2, axis=0, keepdims=True)
    return (u - m) / jnp.sqrt(v + 1e-5) * g + b


def _mlp_core(h_ref, agg_ref, w1_ref, b1_ref, g1_ref, be1_ref, w2_ref,
              b2_ref, eps_ref):
    agg = agg_ref[0, :N] + agg_ref[1, :N]
    t = h_ref[...] * (1.0 + eps_ref[0, 0]) + agg
    u = _dot3(t, w1_ref[...])
    u = u + b1_ref[...]
    u = _bn_cols(u, g1_ref[...], be1_ref[...])
    u = jnp.maximum(u, 0.0)
    o = _dot3(u, w2_ref[...])
    return o + b2_ref[...]


def _mlp_body(h_ref, agg_ref, w1_ref, b1_ref, g1_ref, be1_ref,
              w2_ref, b2_ref, eps_ref, bng_ref, bnb_ref, out_ref):
    o = _mlp_core(h_ref, agg_ref, w1_ref, b1_ref, g1_ref, be1_ref, w2_ref,
                  b2_ref, eps_ref)
    o = _bn_cols(o, bng_ref[...], bnb_ref[...])
    out_ref[...] = jnp.maximum(o, 0.0)


def _mlp(h, agg, w1, b1, g1, be1, w2, b2, eps_i, bng_i, bnb_i):
    return pl.pallas_call(
        _mlp_body,
        out_shape=jax.ShapeDtypeStruct((N, H), jnp.float32),
    )(h, agg, w1, b1.reshape(1, -1), g1.reshape(1, -1), be1.reshape(1, -1),
      w2, b2.reshape(1, -1), eps_i.reshape(1, 1), bng_i.reshape(1, -1),
      bnb_i.reshape(1, -1))


def _last_body(h_ref, agg_ref, w1_ref, b1_ref, g1_ref, be1_ref, w2_ref,
               b2_ref, eps_ref, batch_ref, wo_ref, bo_ref, out_ref):
    o = _mlp_core(h_ref, agg_ref, w1_ref, b1_ref, g1_ref, be1_ref, w2_ref,
                  b2_ref, eps_ref)
    gids = lax.broadcasted_iota(jnp.int32, (G, N), 0)
    onehot = (batch_ref[...] == gids).astype(jnp.float32)
    sums = jnp.dot(onehot, o, preferred_element_type=jnp.float32,
                   precision=lax.Precision.HIGHEST)
    counts = jnp.maximum(jnp.sum(onehot, axis=1, keepdims=True), 1.0)
    pooled = sums / counts
    out_ref[...] = (
        jnp.dot(pooled, wo_ref[...], preferred_element_type=jnp.float32,
                precision=lax.Precision.HIGHEST)
        + bo_ref[...])


def _last_mlp_pool(h, agg, w1, b1, g1, be1, w2, b2, eps_i, batch, W_out,
                   b_out):
    return pl.pallas_call(
        _last_body,
        out_shape=jax.ShapeDtypeStruct((G, W_out.shape[1]), jnp.float32),
    )(h, agg, w1, b1.reshape(1, -1), g1.reshape(1, -1), be1.reshape(1, -1),
      w2, b2.reshape(1, -1), eps_i.reshape(1, 1), batch.reshape(1, N),
      W_out, b_out.reshape(1, -1))


def kernel(x, edge_attr, W_ln, b_ln, W1, b1, g1, be1, W2, b2, eps, bng, bnb,
           W_out, b_out, edge_index, batch):
    src = edge_index[0]
    dst = edge_index[1].reshape(NW, NCH, CH)
    zeros = jnp.zeros((NP, H), jnp.float32)
    h, r = _embed(x, W_ln, b_ln)
    L = W1.shape[0]
    for i in range(L - 1):
        agg = _edge_agg(r if i == 0 else h, src, dst, zeros)
        h = _mlp(h, agg, W1[i], b1[i], g1[i], be1[i], W2[i], b2[i],
                 eps[i], bng[i], bnb[i])
    agg = _edge_agg(h, src, dst, zeros)
    return _last_mlp_pool(h, agg, W1[L - 1], b1[L - 1], g1[L - 1],
                          be1[L - 1], W2[L - 1], b2[L - 1], eps[L - 1],
                          batch, W_out, b_out)


# speedup vs baseline: 1.3417x; 1.0024x over previous
import functools

import jax
import jax.numpy as jnp
from jax import lax
from jax.experimental import pallas as pl
from jax.experimental.pallas import tpu as pltpu
from jax.experimental.pallas import tpu_sc as plsc

N = 10000
E = 320000
H = 128
G = 64

NC = 2
NS = 16
NW = NC * NS
EW = E // NW
CH = 80
NCH = EW // CH
GR = 5
NG = NCH // GR
NP = 10240
RT = NP // NS
ZR = 32


def _edge_body(h_hbm, src_hbm, dst_hbm, zeros_hbm, out_hbm, srcb, dstb,
               bank0, bank1, agg_sh, gsem0, gsem1):
    c = lax.axis_index("c")
    s = lax.axis_index("s")
    wid = s * NC + c

    base = pl.multiple_of(wid * EW, 8)
    pltpu.sync_copy(src_hbm.at[pl.ds(base, EW)], srcb)
    pltpu.sync_copy(dst_hbm.at[wid], dstb)

    def gather(j, bank, gsem):
        off = pl.multiple_of(j * CH, 8)
        pltpu.async_copy(h_hbm.at[srcb.at[pl.ds(off, CH)]], bank, gsem)

    def gwait(bank, gsem):
        pltpu.make_async_copy(h_hbm.at[srcb.at[pl.ds(0, CH)]], bank,
                              gsem).wait()

    def scatter(j, bank):
        pltpu.sync_copy(bank, agg_sh.at[dstb.at[j]], add=True)

    gather(0, bank0, gsem0)
    gather(1, bank1, gsem1)
    row0 = pl.multiple_of(s * RT, 8)
    pltpu.sync_copy(zeros_hbm.at[pl.ds(row0, RT)], agg_sh.at[pl.ds(row0, RT)])
    plsc.subcore_barrier()

    def pipe_step(j2, _):
        ja = j2 * 2
        gwait(bank0, gsem0)
        scatter(ja, bank0)
        gather(ja + 2, bank0, gsem0)
        gwait(bank1, gsem1)
        scatter(ja + 1, bank1)

        @pl.when(j2 < (NCH - 3) // 2)
        def _():
            gather(ja + 3, bank1, gsem1)
        return 0

    lax.fori_loop(0, (NCH - 1) // 2, pipe_step, 0)
    gwait(bank0, gsem0)
    scatter(NCH - 1, bank0)
    plsc.subcore_barrier()

    row0 = pl.multiple_of(s * RT, 8)
    pltpu.sync_copy(agg_sh.at[pl.ds(row0, RT)],
                    out_hbm.at[c].at[pl.ds(row0, RT)])


_edge_agg = pl.kernel(
    _edge_body,
    out_type=jax.ShapeDtypeStruct((NC, NP, H), jnp.float32),
    mesh=plsc.VectorSubcoreMesh(core_axis_name="c", subcore_axis_name="s"),
    scratch_types=[
        pltpu.VMEM((EW,), jnp.int32),
        pltpu.VMEM((NCH, CH), jnp.int32),
        pltpu.VMEM((CH, H), jnp.float32),
        pltpu.VMEM((CH, H), jnp.float32),
        pltpu.VMEM_SHARED((NP, H), jnp.float32),
        pltpu.SemaphoreType.DMA,
        pltpu.SemaphoreType.DMA,
    ],
)


def _embed_body(x_ref, w_ref, b_ref, h_ref, r_ref):
    h = _dot3(x_ref[...], w_ref[...])
    h = h + b_ref[...]
    h_ref[...] = h
    r_ref[...] = jnp.maximum(h, 0.0)


def _embed(x, W_ln, b_ln):
    return pl.pallas_call(
        _embed_body,
        out_shape=[
            jax.ShapeDtypeStruct((N, H), jnp.float32),
            jax.ShapeDtypeStruct((N, H), jnp.float32),
        ],
    )(x, W_ln, b_ln.reshape(1, H))


def _split_hi_lo(a):
    bits = lax.bitcast_convert_type(a, jnp.uint32)
    hi32 = lax.bitcast_convert_type(bits & jnp.uint32(0xFFFF0000),
                                    jnp.float32)
    return hi32.astype(jnp.bfloat16), (a - hi32).astype(jnp.bfloat16)


def _dot3(a, b):
    a_hi, a_lo = _split_hi_lo(a)
    b_hi, b_lo = _split_hi_lo(b)
    f = jnp.float32
    return (jnp.dot(a_hi, b_hi, preferred_element_type=f)
            + jnp.dot(a_hi, b_lo, preferred_element_type=f)
            + jnp.dot(a_lo, b_hi, preferred_element_type=f))


def _bn_cols(u, g, b):
    m = jnp.mean(u, axis=0, keepdims=True)
    v = jnp.mean((u - m) ** 2, axis=0, keepdims=True)
    return (u - m) / jnp.sqrt(v + 1e-5) * g + b


def _mlp_core(h_ref, agg_ref, w1_ref, b1_ref, g1_ref, be1_ref, w2_ref,
              b2_ref, eps_ref):
    agg = agg_ref[0, :N] + agg_ref[1, :N]
    t = h_ref[...] * (1.0 + eps_ref[0, 0]) + agg
    u = _dot3(t, w1_ref[...])
    u = u + b1_ref[...]
    u = _bn_cols(u, g1_ref[...], be1_ref[...])
    u = jnp.maximum(u, 0.0)
    o = _dot3(u, w2_ref[...])
    return o + b2_ref[...]


def _mlp_body(h_ref, agg_ref, w1_ref, b1_ref, g1_ref, be1_ref,
              w2_ref, b2_ref, eps_ref, bng_ref, bnb_ref, out_ref):
    o = _mlp_core(h_ref, agg_ref, w1_ref, b1_ref, g1_ref, be1_ref, w2_ref,
                  b2_ref, eps_ref)
    o = _bn_cols(o, bng_ref[...], bnb_ref[...])
    out_ref[...] = jnp.maximum(o, 0.0)


def _mlp(h, agg, w1, b1, g1, be1, w2, b2, eps_i, bng_i, bnb_i):
    return pl.pallas_call(
        _mlp_body,
        out_shape=jax.ShapeDtypeStruct((N, H), jnp.float32),
    )(h, agg, w1, b1.reshape(1, -1), g1.reshape(1, -1), be1.reshape(1, -1),
      w2, b2.reshape(1, -1), eps_i.reshape(1, 1), bng_i.reshape(1, -1),
      bnb_i.reshape(1, -1))


def _last_body(h_ref, agg_ref, w1_ref, b1_ref, g1_ref, be1_ref, w2_ref,
               b2_ref, eps_ref, batch_ref, wo_ref, bo_ref, out_ref):
    o = _mlp_core(h_ref, agg_ref, w1_ref, b1_ref, g1_ref, be1_ref, w2_ref,
                  b2_ref, eps_ref)
    gids = lax.broadcasted_iota(jnp.int32, (G, N), 0)
    onehot = (batch_ref[...] == gids).astype(jnp.float32)
    sums = jnp.dot(onehot, o, preferred_element_type=jnp.float32,
                   precision=lax.Precision.HIGHEST)
    counts = jnp.maximum(jnp.sum(onehot, axis=1, keepdims=True), 1.0)
    pooled = sums / counts
    out_ref[...] = (
        jnp.dot(pooled, wo_ref[...], preferred_element_type=jnp.float32,
                precision=lax.Precision.HIGHEST)
        + bo_ref[...])


def _last_mlp_pool(h, agg, w1, b1, g1, be1, w2, b2, eps_i, batch, W_out,
                   b_out):
    return pl.pallas_call(
        _last_body,
        out_shape=jax.ShapeDtypeStruct((G, W_out.shape[1]), jnp.float32),
    )(h, agg, w1, b1.reshape(1, -1), g1.reshape(1, -1), be1.reshape(1, -1),
      w2, b2.reshape(1, -1), eps_i.reshape(1, 1), batch.reshape(1, N),
      W_out, b_out.reshape(1, -1))


def kernel(x, edge_attr, W_ln, b_ln, W1, b1, g1, be1, W2, b2, eps, bng, bnb,
           W_out, b_out, edge_index, batch):
    src = edge_index[0]
    dst = edge_index[1].reshape(NW, NCH, CH)
    zeros = jnp.zeros((NP, H), jnp.float32)
    h, r = _embed(x, W_ln, b_ln)
    L = W1.shape[0]
    for i in range(L - 1):
        agg = _edge_agg(r if i == 0 else h, src, dst, zeros)
        h = _mlp(h, agg, W1[i], b1[i], g1[i], be1[i], W2[i], b2[i],
                 eps[i], bng[i], bnb[i])
    agg = _edge_agg(h, src, dst, zeros)
    return _last_mlp_pool(h, agg, W1[L - 1], b1[L - 1], g1[L - 1],
                          be1[L - 1], W2[L - 1], b2[L - 1], eps[L - 1],
                          batch, W_out, b_out)


# speedup vs baseline: 1.4497x; 1.0805x over previous
import functools

import jax
import jax.numpy as jnp
from jax import lax
from jax.experimental import pallas as pl
from jax.experimental.pallas import tpu as pltpu
from jax.experimental.pallas import tpu_sc as plsc

N = 10000
E = 320000
H = 128
G = 64

NC = 2
NS = 16
NW = NC * NS
EW = E // NW
CH = 80
NCH = EW // CH
GR = 5
NG = NCH // GR
NP = 10240
RT = NP // NS
ZR = 32


def _edge_body(h_hbm, src_hbm, dst_hbm, zeros_hbm, out_hbm, srcb, dstb,
               bank0, bank1, agg_sh, gsem0, gsem1):
    c = lax.axis_index("c")
    s = lax.axis_index("s")
    wid = s * NC + c

    base = pl.multiple_of(wid * EW, 8)
    pltpu.sync_copy(src_hbm.at[pl.ds(base, EW)], srcb)
    pltpu.sync_copy(dst_hbm.at[wid], dstb)

    def gather(j, bank, gsem):
        off = pl.multiple_of(j * CH, 8)
        pltpu.async_copy(h_hbm.at[srcb.at[pl.ds(off, CH)]], bank, gsem)

    def gwait(bank, gsem):
        pltpu.make_async_copy(h_hbm.at[srcb.at[pl.ds(0, CH)]], bank,
                              gsem).wait()

    def scatter(j, bank):
        pltpu.sync_copy(bank, agg_sh.at[dstb.at[j]], add=True)

    gather(0, bank0, gsem0)
    gather(1, bank1, gsem1)
    row0 = pl.multiple_of(s * RT, 8)
    pltpu.sync_copy(zeros_hbm.at[pl.ds(row0, RT)], agg_sh.at[pl.ds(row0, RT)])
    plsc.subcore_barrier()

    def pipe_step(j2, _):
        ja = j2 * 2
        gwait(bank0, gsem0)
        scatter(ja, bank0)
        gather(ja + 2, bank0, gsem0)
        gwait(bank1, gsem1)
        scatter(ja + 1, bank1)

        @pl.when(j2 < (NCH - 3) // 2)
        def _():
            gather(ja + 3, bank1, gsem1)
        return 0

    lax.fori_loop(0, (NCH - 1) // 2, pipe_step, 0)
    gwait(bank0, gsem0)
    scatter(NCH - 1, bank0)
    plsc.subcore_barrier()

    row0 = pl.multiple_of(s * RT, 8)
    pltpu.sync_copy(agg_sh.at[pl.ds(row0, RT)],
                    out_hbm.at[c].at[pl.ds(row0, RT)])


_edge_agg = pl.kernel(
    _edge_body,
    out_type=jax.ShapeDtypeStruct((NC, NP, H), jnp.float32),
    mesh=plsc.VectorSubcoreMesh(core_axis_name="c", subcore_axis_name="s"),
    scratch_types=[
        pltpu.VMEM((EW,), jnp.int32),
        pltpu.VMEM((NCH, CH), jnp.int32),
        pltpu.VMEM((CH, H), jnp.float32),
        pltpu.VMEM((CH, H), jnp.float32),
        pltpu.VMEM_SHARED((NP, H), jnp.float32),
        pltpu.SemaphoreType.DMA,
        pltpu.SemaphoreType.DMA,
    ],
)


def _embed_body(x_ref, w_ref, b_ref, h_ref, r_ref):
    h = _dot_mimic(x_ref[...], w_ref[...])
    h = h + b_ref[...]
    h_ref[...] = h
    r_ref[...] = jnp.maximum(h, 0.0)


def _embed(x, W_ln, b_ln):
    return pl.pallas_call(
        _embed_body,
        out_shape=[
            jax.ShapeDtypeStruct((N, H), jnp.float32),
            jax.ShapeDtypeStruct((N, H), jnp.float32),
        ],
    )(x, W_ln, b_ln.reshape(1, H))


def _dot_mimic(a, b):
    return jnp.dot(a.astype(jnp.bfloat16), b.astype(jnp.bfloat16),
                   preferred_element_type=jnp.float32)


def _bn_cols(u, g, b):
    m = jnp.mean(u, axis=0, keepdims=True)
    v = jnp.mean((u - m) ** 2, axis=0, keepdims=True)
    return (u - m) / jnp.sqrt(v + 1e-5) * g + b


def _mlp_core(h_ref, agg_ref, w1_ref, b1_ref, g1_ref, be1_ref, w2_ref,
              b2_ref, eps_ref):
    agg = agg_ref[0, :N] + agg_ref[1, :N]
    t = h_ref[...] * (1.0 + eps_ref[0, 0]) + agg
    u = _dot_mimic(t, w1_ref[...])
    u = u + b1_ref[...]
    u = _bn_cols(u, g1_ref[...], be1_ref[...])
    u = jnp.maximum(u, 0.0)
    o = _dot_mimic(u, w2_ref[...])
    return o + b2_ref[...]


def _mlp_body(h_ref, agg_ref, w1_ref, b1_ref, g1_ref, be1_ref,
              w2_ref, b2_ref, eps_ref, bng_ref, bnb_ref, out_ref):
    o = _mlp_core(h_ref, agg_ref, w1_ref, b1_ref, g1_ref, be1_ref, w2_ref,
                  b2_ref, eps_ref)
    o = _bn_cols(o, bng_ref[...], bnb_ref[...])
    out_ref[...] = jnp.maximum(o, 0.0)


def _mlp(h, agg, w1, b1, g1, be1, w2, b2, eps_i, bng_i, bnb_i):
    return pl.pallas_call(
        _mlp_body,
        out_shape=jax.ShapeDtypeStruct((N, H), jnp.float32),
    )(h, agg, w1, b1.reshape(1, -1), g1.reshape(1, -1), be1.reshape(1, -1),
      w2, b2.reshape(1, -1), eps_i.reshape(1, 1), bng_i.reshape(1, -1),
      bnb_i.reshape(1, -1))


def _last_body(h_ref, agg_ref, w1_ref, b1_ref, g1_ref, be1_ref, w2_ref,
               b2_ref, eps_ref, batch_ref, wo_ref, bo_ref, out_ref):
    o = _mlp_core(h_ref, agg_ref, w1_ref, b1_ref, g1_ref, be1_ref, w2_ref,
                  b2_ref, eps_ref)
    gids = lax.broadcasted_iota(jnp.int32, (G, N), 0)
    onehot = (batch_ref[...] == gids).astype(jnp.float32)
    sums = jnp.dot(onehot, o, preferred_element_type=jnp.float32,
                   precision=lax.Precision.HIGHEST)
    counts = jnp.maximum(jnp.sum(onehot, axis=1, keepdims=True), 1.0)
    pooled = sums / counts
    out_ref[...] = (
        jnp.dot(pooled, wo_ref[...], preferred_element_type=jnp.float32)
        + bo_ref[...])


def _last_mlp_pool(h, agg, w1, b1, g1, be1, w2, b2, eps_i, batch, W_out,
                   b_out):
    return pl.pallas_call(
        _last_body,
        out_shape=jax.ShapeDtypeStruct((G, W_out.shape[1]), jnp.float32),
    )(h, agg, w1, b1.reshape(1, -1), g1.reshape(1, -1), be1.reshape(1, -1),
      w2, b2.reshape(1, -1), eps_i.reshape(1, 1), batch.reshape(1, N),
      W_out, b_out.reshape(1, -1))


def kernel(x, edge_attr, W_ln, b_ln, W1, b1, g1, be1, W2, b2, eps, bng, bnb,
           W_out, b_out, edge_index, batch):
    src = edge_index[0]
    dst = edge_index[1].reshape(NW, NCH, CH)
    zeros = jnp.zeros((NP, H), jnp.float32)
    h, r = _embed(x, W_ln, b_ln)
    L = W1.shape[0]
    for i in range(L - 1):
        agg = _edge_agg(r if i == 0 else h, src, dst, zeros)
        h = _mlp(h, agg, W1[i], b1[i], g1[i], be1[i], W2[i], b2[i],
                 eps[i], bng[i], bnb[i])
    agg = _edge_agg(h, src, dst, zeros)
    return _last_mlp_pool(h, agg, W1[L - 1], b1[L - 1], g1[L - 1],
                          be1[L - 1], W2[L - 1], b2[L - 1], eps[L - 1],
                          batch, W_out, b_out)
